# Initial kernel scaffold; baseline (speedup 1.0000x reference)
import jax
import jax.numpy as jnp
from jax.experimental import pallas as pl


def kernel(Xg, Xs, Xa, edge_index, Act_pos, batch_g, params):
    raise NotImplementedError("write your pallas kernel here")


# speedup vs baseline: 1.2039x; 1.2039x over previous
import jax
import jax.numpy as jnp
from jax.experimental import pallas as pl
from jax.experimental.pallas import tpu as pltpu

H = 128
B = 128
S = 64
F32 = jnp.float32


def _leaky(x, s):
    return jnp.where(x >= 0, x, s * x)


def _dot(a, b):
    return jnp.dot(a, b, preferred_element_type=jnp.float32)


def _gates(gi, gh, h):
    r = jax.nn.sigmoid(gi[:, :H] + gh[:, :H])
    z = jax.nn.sigmoid(gi[:, H:2 * H] + gh[:, H:2 * H])
    n = jnp.tanh(gi[:, 2 * H:] + r * gh[:, 2 * H:])
    return (1.0 - z) * n + z * h


def _ln(x, g, b):
    m = jnp.mean(x, axis=1, keepdims=True)
    v = jnp.mean((x - m) ** 2, axis=1, keepdims=True)
    return (x - m) / jnp.sqrt(v + 1e-5) * g + b


def _enc_body(x_ref,
              w1fi, w1fh, b1fi, b1fh,
              w1bi, w1bh, b1bi, b1bh,
              w2fi, w2fh, b2fi, b2fh,
              w2bi, w2bh, b2bi, b2bh,
              out_ref, hf_ref, hb_ref,
              y1_ref, gif_ref, gib_ref):
    CH = 2048
    for c in range(S * B // CH):
        sl = slice(c * CH, (c + 1) * CH)
        xc = x_ref[sl, :]
        gif_ref[sl, :] = _dot(xc, w1fi[:]) + b1fi[:]
        gib_ref[sl, :] = _dot(xc, w1bi[:]) + b1bi[:]

    def layer(whf, bhf, whb, bhb, dst_ref):
        def step(t, carry):
            hf, hb = carry
            tb = S - 1 - t
            ghf = _dot(hf, whf[:]) + bhf[:]
            hfn = _gates(gif_ref[pl.ds(t * B, B), :], ghf, hf)
            dst_ref[pl.ds(t * B, B), 0:H] = hfn
            ghb = _dot(hb, whb[:]) + bhb[:]
            hbn = _gates(gib_ref[pl.ds(tb * B, B), :], ghb, hb)
            dst_ref[pl.ds(tb * B, B), H:2 * H] = hbn
            return hfn, hbn

        h0 = jnp.zeros((B, H), F32)
        return jax.lax.fori_loop(0, S, step, (h0, h0))

    layer(w1fh, b1fh, w1bh, b1bh, y1_ref)
    for c in range(S * B // CH):
        sl = slice(c * CH, (c + 1) * CH)
        yc = y1_ref[sl, :]
        gif_ref[sl, :] = _dot(yc, w2fi[:]) + b2fi[:]
        gib_ref[sl, :] = _dot(yc, w2bi[:]) + b2bi[:]
    hf, hb = layer(w2fh, b2fh, w2bh, b2bh, out_ref)
    hf_ref[:] = hf
    hb_ref[:] = hb


def _encoder(x_tm, enc):
    args = [x_tm]
    for l in range(2):
        for d in ("f", "b"):
            w = enc[l][d]
            args += [w["Wi"].T, w["Wh"].T, w["bi"][None, :], w["bh"][None, :]]
    return pl.pallas_call(
        _enc_body,
        out_shape=[jax.ShapeDtypeStruct((S * B, 2 * H), F32),
                   jax.ShapeDtypeStruct((B, H), F32),
                   jax.ShapeDtypeStruct((B, H), F32)],
        scratch_shapes=[pltpu.VMEM((S * B, 2 * H), F32),
                        pltpu.VMEM((S * B, 3 * H), F32),
                        pltpu.VMEM((S * B, 3 * H), F32)],
    )(*args)


def _head_body(z_ref, h_ref, clng, clnb, cw, cb, vlng, vlnb, vw, vb,
               ctx_ref, rep_ref):
    zn = _ln(z_ref[:], clng[:], clnb[:])
    ctx_ref[:] = _leaky(_dot(zn, cw[:]) + cb[:], 0.01)
    CH = 2048
    for c in range(S * B // CH):
        sl = slice(c * CH, (c + 1) * CH)
        hn = _ln(h_ref[sl, :], vlng[:], vlnb[:])
        rep_ref[sl, :] = _leaky(_dot(hn, vw[:]) + vb[:], 0.01)


def _head(z, h_tm, p):
    return pl.pallas_call(
        _head_body,
        out_shape=[jax.ShapeDtypeStruct((B, 3 * H), F32),
                   jax.ShapeDtypeStruct((S * B, 2 * H), F32)],
    )(z, h_tm,
      p["ctx_ln_g"][None, :], p["ctx_ln_b"][None, :],
      p["ctx_W"], p["ctx_b"][None, :],
      p["vec_ln_g"][None, :], p["vec_ln_b"][None, :],
      p["vec_W"], p["vec_b"][None, :])


def _dec_body(ctx_ref, rep_ref, emb_ref,
              w0i, w0h, b0i, b0h, w1i, w1h, b1i, b1h, lw, lb,
              out_ref, gi0_ref, y_ref):
    gi0_ref[0:B, :] = _dot(ctx_ref[:], w0i[:]) + b0i[:]
    CH = 2016
    for c in range(4):
        sl = slice(c * CH, (c + 1) * CH)
        osl = slice(B + c * CH, B + (c + 1) * CH)
        gi0_ref[osl, :] = (_dot(rep_ref[sl, :], w0i[0:2 * H, :])
                           + _dot(emb_ref[sl, :], w0i[2 * H:3 * H, :])
                           + b0i[:])

    def step(t, carry):
        h0, h1 = carry
        gh0 = _dot(h0, w0h[:]) + b0h[:]
        h0n = _gates(gi0_ref[pl.ds(t * B, B), :], gh0, h0)
        gi1 = _dot(h0n, w1i[:]) + b1i[:]
        gh1 = _dot(h1, w1h[:]) + b1h[:]
        h1n = _gates(gi1, gh1, h1)
        y_ref[pl.ds(t * B, B), :] = h1n
        return h0n, h1n

    hz = jnp.zeros((B, H), F32)
    jax.lax.fori_loop(0, S, step, (hz, hz))
    for c in range(4):
        sl = slice(c * 2048, (c + 1) * 2048)
        out_ref[sl, :] = _dot(y_ref[sl, :], lw[:]) + lb[:]


def _decoder(ctx, rep, emb, d):
    l0, l1 = d["layers"]
    dout = d["lin_W"].shape[1]
    return pl.pallas_call(
        _dec_body,
        out_shape=jax.ShapeDtypeStruct((S * B, dout), F32),
        scratch_shapes=[pltpu.VMEM((S * B, 3 * H), F32),
                        pltpu.VMEM((S * B, H), F32)],
    )(ctx, rep, emb,
      l0["Wi"].T, l0["Wh"].T, l0["bi"][None, :], l0["bh"][None, :],
      l1["Wi"].T, l1["Wh"].T, l1["bi"][None, :], l1["bh"][None, :],
      d["lin_W"], d["lin_b"][None, :])


def _gat_j(x, src, dst, W, asrc, adst, bias):
    N = x.shape[0]
    xp = (x @ W).reshape(N, 2, H)
    a_s = jnp.sum(xp * asrc[None, :, :], axis=-1)
    a_d = jnp.sum(xp * adst[None, :, :], axis=-1)
    e = _leaky(a_s[src] + a_d[dst], 0.2)
    emax = jax.ops.segment_max(e, dst, num_segments=N)
    ee = jnp.exp(e - emax[dst])
    den = jax.ops.segment_sum(ee, dst, num_segments=N)
    alpha = ee / (den[dst] + 1e-16)
    out = jax.ops.segment_sum(xp[src] * alpha[:, :, None], dst, num_segments=N)
    return out.reshape(N, 2 * H) + bias


def _posenc(seq_len, d):
    positions = jnp.arange(seq_len, dtype=F32)[:, None]
    div = jnp.exp(-jnp.log(10000.0) * jnp.arange(0, d, 2, dtype=F32) / d)
    ang = positions * div[None, :]
    pe = jnp.zeros((seq_len, d), dtype=F32)
    pe = pe.at[:, 0::2].set(jnp.sin(ang))
    pe = pe.at[:, 1::2].set(jnp.cos(ang))
    return pe


def kernel(Xg, Xs, Xa, edge_index, Act_pos, batch_g, params):
    p = params
    N = Xg.shape[0]
    loop = jnp.arange(N)
    src = jnp.concatenate([edge_index[0], loop])
    dst = jnp.concatenate([edge_index[1], loop])
    x_emb = p["acts_emb"][Xg[:, 0]]
    h1 = _leaky(_gat_j(x_emb, src, dst, p["gat1_W"], p["gat1_asrc"],
                       p["gat1_adst"], p["gat1_b"]), 0.01)
    h2 = _leaky(_gat_j(h1, src, dst, p["gat2_W"], p["gat2_asrc"],
                       p["gat2_adst"], p["gat2_b"]), 0.01)
    out_g = h2 @ p["proj_W"] + p["proj_b"]
    sums = jax.ops.segment_sum(out_g, batch_g, num_segments=B)
    counts = jax.ops.segment_sum(jnp.ones((N,), F32), batch_g, num_segments=B)
    z_g = sums / jnp.maximum(counts, 1.0)[:, None]

    ids0 = Xs[:, :, 0].T
    ids1 = Xs[:, :, 1].T
    x_tm = jnp.concatenate([p["attr_emb_0"][ids0], p["attr_emb_1"][ids1]],
                           axis=-1).reshape(S * B, 2 * H)
    out_s_tm, hf, hb = _encoder(x_tm, p["enc_gru"])
    z = jnp.concatenate([z_g, hf, hb], axis=1)

    ap = Act_pos.T
    valid = ap >= 0
    gathered = out_g[jnp.clip(ap, 0, N - 1)]
    mapped = jnp.where(valid[:, :, None], gathered, 0.0)
    pos_tm = _posenc(S, H)[:, None, :] * valid[:, :, None].astype(F32)
    h_tm = jnp.concatenate([out_s_tm.reshape(S, B, 2 * H), mapped, pos_tm],
                           axis=-1).reshape(S * B, 4 * H)
    ctx, rep = _head(z, h_tm, p)

    outs = []
    for i in range(3):
        ids = (Xa if i == 0 else Xs[:, :, i - 1]).T[:S - 1]
        emb = p["emb_tabs"][i][ids].reshape((S - 1) * B, H)
        y = _decoder(ctx, rep, emb, p["decs"][i])
        outs.append(jnp.swapaxes(y.reshape(S, B, -1), 0, 1))
    return tuple(outs)


# speedup vs baseline: 1.2161x; 1.0101x over previous
import functools

import jax
import jax.numpy as jnp
from jax import lax
from jax.experimental import pallas as pl
from jax.experimental.pallas import tpu as pltpu
from jax.experimental.pallas import tpu_sc as plsc

H = 128
B = 128
S = 64
N_NODES = 4096
F32 = jnp.float32


def _leaky(x, s):
    return jnp.where(x >= 0, x, s * x)


def _dot(a, b):
    return jnp.dot(a, b, preferred_element_type=jnp.float32)


def _gates(gi, gh, h):
    r = jax.nn.sigmoid(gi[:, :H] + gh[:, :H])
    z = jax.nn.sigmoid(gi[:, H:2 * H] + gh[:, H:2 * H])
    n = jnp.tanh(gi[:, 2 * H:] + r * gh[:, 2 * H:])
    return (1.0 - z) * n + z * h


def _ln(x, g, b):
    m = jnp.mean(x, axis=1, keepdims=True)
    v = jnp.mean((x - m) ** 2, axis=1, keepdims=True)
    return (x - m) / jnp.sqrt(v + 1e-5) * g + b


def _enc_body(x_ref,
              w1fi, w1fh, b1fi, b1fh,
              w1bi, w1bh, b1bi, b1bh,
              w2fi, w2fh, b2fi, b2fh,
              w2bi, w2bh, b2bi, b2bh,
              out_ref, hf_ref, hb_ref,
              y1_ref, gif_ref, gib_ref):
    CH = 2048
    for c in range(S * B // CH):
        sl = slice(c * CH, (c + 1) * CH)
        xc = x_ref[sl, :]
        gif_ref[sl, :] = _dot(xc, w1fi[:]) + b1fi[:]
        gib_ref[sl, :] = _dot(xc, w1bi[:]) + b1bi[:]

    def layer(whf, bhf, whb, bhb, dst_ref):
        def step(t, carry):
            hf, hb = carry
            tb = S - 1 - t
            ghf = _dot(hf, whf[:]) + bhf[:]
            hfn = _gates(gif_ref[pl.ds(t * B, B), :], ghf, hf)
            dst_ref[pl.ds(t * B, B), 0:H] = hfn
            ghb = _dot(hb, whb[:]) + bhb[:]
            hbn = _gates(gib_ref[pl.ds(tb * B, B), :], ghb, hb)
            dst_ref[pl.ds(tb * B, B), H:2 * H] = hbn
            return hfn, hbn

        h0 = jnp.zeros((B, H), F32)
        return jax.lax.fori_loop(0, S, step, (h0, h0))

    layer(w1fh, b1fh, w1bh, b1bh, y1_ref)
    for c in range(S * B // CH):
        sl = slice(c * CH, (c + 1) * CH)
        yc = y1_ref[sl, :]
        gif_ref[sl, :] = _dot(yc, w2fi[:]) + b2fi[:]
        gib_ref[sl, :] = _dot(yc, w2bi[:]) + b2bi[:]
    hf, hb = layer(w2fh, b2fh, w2bh, b2bh, out_ref)
    hf_ref[:] = hf
    hb_ref[:] = hb


def _encoder(x_tm, enc):
    args = [x_tm]
    for l in range(2):
        for d in ("f", "b"):
            w = enc[l][d]
            args += [w["Wi"].T, w["Wh"].T, w["bi"][None, :], w["bh"][None, :]]
    return pl.pallas_call(
        _enc_body,
        out_shape=[jax.ShapeDtypeStruct((S * B, 2 * H), F32),
                   jax.ShapeDtypeStruct((B, H), F32),
                   jax.ShapeDtypeStruct((B, H), F32)],
        scratch_shapes=[pltpu.VMEM((S * B, 2 * H), F32),
                        pltpu.VMEM((S * B, 3 * H), F32),
                        pltpu.VMEM((S * B, 3 * H), F32)],
    )(*args)


def _head_body(sums_ref, cnts_ref, hf_ref, hb_ref, h_ref,
               clng, clnb, cw, cb, vlng, vlnb, vw, vb,
               ctx_ref, rep_ref):
    sums = sums_ref[0] + sums_ref[1]
    cnts = cnts_ref[0] + cnts_ref[1]
    z_g = sums / jnp.maximum(cnts, 1.0)
    z = jnp.concatenate([z_g, hf_ref[:], hb_ref[:]], axis=1)
    zn = _ln(z, clng[:], clnb[:])
    ctx_ref[:] = _leaky(_dot(zn, cw[:]) + cb[:], 0.01)
    CH = 2048
    for c in range(S * B // CH):
        sl = slice(c * CH, (c + 1) * CH)
        hn = _ln(h_ref[sl, :], vlng[:], vlnb[:])
        rep_ref[sl, :] = _leaky(_dot(hn, vw[:]) + vb[:], 0.01)


def _head(sums_p, cnts_p, hf, hb, h_tm, p):
    return pl.pallas_call(
        _head_body,
        out_shape=[jax.ShapeDtypeStruct((B, 3 * H), F32),
                   jax.ShapeDtypeStruct((S * B, 2 * H), F32)],
    )(sums_p, cnts_p, hf, hb, h_tm,
      p["ctx_ln_g"][None, :], p["ctx_ln_b"][None, :],
      p["ctx_W"], p["ctx_b"][None, :],
      p["vec_ln_g"][None, :], p["vec_ln_b"][None, :],
      p["vec_W"], p["vec_b"][None, :])


def _dec_body(ctx_ref, rep_ref, emb_ref,
              w0i, w0h, b0i, b0h, w1i, w1h, b1i, b1h, lw, lb,
              out_ref, gi0_ref, y_ref):
    gi0_ref[0:B, :] = _dot(ctx_ref[:], w0i[:]) + b0i[:]
    CH = 2016
    for c in range(4):
        sl = slice(c * CH, (c + 1) * CH)
        osl = slice(B + c * CH, B + (c + 1) * CH)
        gi0_ref[osl, :] = (_dot(rep_ref[sl, :], w0i[0:2 * H, :])
                           + _dot(emb_ref[sl, :], w0i[2 * H:3 * H, :])
                           + b0i[:])

    def step(t, carry):
        h0, h1 = carry
        gh0 = _dot(h0, w0h[:]) + b0h[:]
        h0n = _gates(gi0_ref[pl.ds(t * B, B), :], gh0, h0)
        gi1 = _dot(h0n, w1i[:]) + b1i[:]
        gh1 = _dot(h1, w1h[:]) + b1h[:]
        h1n = _gates(gi1, gh1, h1)
        y_ref[pl.ds(t * B, B), :] = h1n
        return h0n, h1n

    hz = jnp.zeros((B, H), F32)
    jax.lax.fori_loop(0, S, step, (hz, hz))
    for c in range(4):
        sl = slice(c * 2048, (c + 1) * 2048)
        out_ref[sl, :] = _dot(y_ref[sl, :], lw[:]) + lb[:]


def _decoder(ctx, rep, emb, d):
    l0, l1 = d["layers"]
    dout = d["lin_W"].shape[1]
    return pl.pallas_call(
        _dec_body,
        out_shape=jax.ShapeDtypeStruct((S * B, dout), F32),
        scratch_shapes=[pltpu.VMEM((S * B, 3 * H), F32),
                        pltpu.VMEM((S * B, H), F32)],
    )(ctx, rep, emb,
      l0["Wi"].T, l0["Wh"].T, l0["bi"][None, :], l0["bh"][None, :],
      l1["Wi"].T, l1["Wh"].T, l1["bi"][None, :], l1["bh"][None, :],
      d["lin_W"], d["lin_b"][None, :])


def _sc_gather_body(og_ref, ap_ref, bg_ref, ones_ref, zeros_ref,
                    mapped_ref, sums_ref, cnts_ref,
                    ap_v, idx_v, rows_v, gsl_v, bidx_v, ones_v,
                    sh_sums, sh_cnts, sem):
    c = lax.axis_index("c")
    s = lax.axis_index("s")
    nc = 2
    wid = s * nc + c

    @pl.when(s == 0)
    def _zero():
        pltpu.sync_copy(zeros_ref, sh_sums)
        pltpu.sync_copy(zeros_ref, sh_cnts)

    plsc.subcore_barrier()

    pltpu.sync_copy(ap_ref.at[pl.ds(wid * 256, 256)], ap_v)
    for j in range(2):
        for i in range(8):
            v = ap_v[pl.ds(j * 128 + i * 16, 16)]
            idx_v[j, pl.ds(i * 16, 16)] = jnp.where(
                v >= 0, v, jnp.full((16,), N_NODES, jnp.int32))
    for j in range(2):
        pltpu.async_copy(og_ref.at[idx_v.at[j]],
                         rows_v.at[pl.ds(j * 128, 128)], sem).wait()
    pltpu.sync_copy(rows_v, mapped_ref.at[pl.ds(wid * 256, 256)])

    pltpu.sync_copy(og_ref.at[pl.ds(wid * 128, 128)], gsl_v)
    pltpu.sync_copy(bg_ref.at[pl.ds(wid * 128, 128)], bidx_v)
    pltpu.sync_copy(ones_ref, ones_v)
    pltpu.sync_copy(gsl_v, sh_sums.at[bidx_v], add=True)
    pltpu.sync_copy(ones_v, sh_cnts.at[bidx_v], add=True)
    plsc.subcore_barrier()

    @pl.when(s == 0)
    def _flush():
        pltpu.sync_copy(sh_sums, sums_ref.at[c])
        pltpu.sync_copy(sh_cnts, cnts_ref.at[c])


def _sc_gather_segsum(og_ext, ap_flat, batch_g):
    ones = jnp.ones((B, H), F32)
    zeros = jnp.zeros((B, H), F32)
    mesh = plsc.VectorSubcoreMesh(core_axis_name="c", subcore_axis_name="s")
    fn = pl.kernel(
        _sc_gather_body,
        mesh=mesh,
        out_type=[jax.ShapeDtypeStruct((S * B, H), F32),
                  jax.ShapeDtypeStruct((2, B, H), F32),
                  jax.ShapeDtypeStruct((2, B, H), F32)],
        scratch_types=[pltpu.VMEM((256,), jnp.int32),
                       pltpu.VMEM((2, 128), jnp.int32),
                       pltpu.VMEM((256, H), F32),
                       pltpu.VMEM((128, H), F32),
                       pltpu.VMEM((128,), jnp.int32),
                       pltpu.VMEM((B, H), F32),
                       pltpu.VMEM_SHARED((B, H), F32),
                       pltpu.VMEM_SHARED((B, H), F32),
                       pltpu.SemaphoreType.DMA],
    )
    return fn(og_ext, ap_flat, batch_g, ones, zeros)


def _gat_j(x, src, dst, W, asrc, adst, bias):
    N = x.shape[0]
    xp = (x @ W).reshape(N, 2, H)
    a_s = jnp.sum(xp * asrc[None, :, :], axis=-1)
    a_d = jnp.sum(xp * adst[None, :, :], axis=-1)
    e = _leaky(a_s[src] + a_d[dst], 0.2)
    emax = jax.ops.segment_max(e, dst, num_segments=N)
    ee = jnp.exp(e - emax[dst])
    den = jax.ops.segment_sum(ee, dst, num_segments=N)
    alpha = ee / (den[dst] + 1e-16)
    out = jax.ops.segment_sum(xp[src] * alpha[:, :, None], dst, num_segments=N)
    return out.reshape(N, 2 * H) + bias


def _posenc(seq_len, d):
    positions = jnp.arange(seq_len, dtype=F32)[:, None]
    div = jnp.exp(-jnp.log(10000.0) * jnp.arange(0, d, 2, dtype=F32) / d)
    ang = positions * div[None, :]
    pe = jnp.zeros((seq_len, d), dtype=F32)
    pe = pe.at[:, 0::2].set(jnp.sin(ang))
    pe = pe.at[:, 1::2].set(jnp.cos(ang))
    return pe


def kernel(Xg, Xs, Xa, edge_index, Act_pos, batch_g, params):
    p = params
    N = Xg.shape[0]
    loop = jnp.arange(N)
    src = jnp.concatenate([edge_index[0], loop])
    dst = jnp.concatenate([edge_index[1], loop])
    x_emb = p["acts_emb"][Xg[:, 0]]
    h1 = _leaky(_gat_j(x_emb, src, dst, p["gat1_W"], p["gat1_asrc"],
                       p["gat1_adst"], p["gat1_b"]), 0.01)
    h2 = _leaky(_gat_j(h1, src, dst, p["gat2_W"], p["gat2_asrc"],
                       p["gat2_adst"], p["gat2_b"]), 0.01)
    out_g = h2 @ p["proj_W"] + p["proj_b"]
    og_ext = jnp.concatenate([out_g, jnp.zeros((8, H), F32)], axis=0)

    ap = Act_pos.T
    mapped_f, sums_p, cnts_p = _sc_gather_segsum(
        og_ext, ap.reshape(S * B).astype(jnp.int32), batch_g.astype(jnp.int32))
    mapped = mapped_f.reshape(S, B, H)

    ids0 = Xs[:, :, 0].T
    ids1 = Xs[:, :, 1].T
    x_tm = jnp.concatenate([p["attr_emb_0"][ids0], p["attr_emb_1"][ids1]],
                           axis=-1).reshape(S * B, 2 * H)
    out_s_tm, hf, hb = _encoder(x_tm, p["enc_gru"])

    valid = ap >= 0
    pos_tm = _posenc(S, H)[:, None, :] * valid[:, :, None].astype(F32)
    h_tm = jnp.concatenate([out_s_tm.reshape(S, B, 2 * H), mapped, pos_tm],
                           axis=-1).reshape(S * B, 4 * H)
    ctx, rep = _head(sums_p, cnts_p, hf, hb, h_tm, p)

    outs = []
    for i in range(3):
        ids = (Xa if i == 0 else Xs[:, :, i - 1]).T[:S - 1]
        emb = p["emb_tabs"][i][ids].reshape((S - 1) * B, H)
        y = _decoder(ctx, rep, emb, p["decs"][i])
        outs.append(jnp.swapaxes(y.reshape(S, B, -1), 0, 1))
    return tuple(outs)


# speedup vs baseline: 7.8805x; 6.4802x over previous
import functools

import jax
import jax.numpy as jnp
from jax import lax
from jax.experimental import pallas as pl
from jax.experimental.pallas import tpu as pltpu
from jax.experimental.pallas import tpu_sc as plsc

H = 128
B = 128
S = 64
N_NODES = 4096
F32 = jnp.float32


def _leaky(x, s):
    return jnp.where(x >= 0, x, s * x)


def _dot(a, b):
    return jnp.dot(a, b, preferred_element_type=jnp.float32)


def _gates(gi, gh, h):
    r = jax.nn.sigmoid(gi[:, :H] + gh[:, :H])
    z = jax.nn.sigmoid(gi[:, H:2 * H] + gh[:, H:2 * H])
    n = jnp.tanh(gi[:, 2 * H:] + r * gh[:, 2 * H:])
    return (1.0 - z) * n + z * h


def _ln(x, g, b):
    m = jnp.mean(x, axis=1, keepdims=True)
    v = jnp.mean((x - m) ** 2, axis=1, keepdims=True)
    return (x - m) / jnp.sqrt(v + 1e-5) * g + b


def _enc_body(x_ref,
              w1fi, w1fh, b1fi, b1fh,
              w1bi, w1bh, b1bi, b1bh,
              w2fi, w2fh, b2fi, b2fh,
              w2bi, w2bh, b2bi, b2bh,
              out_ref, hf_ref, hb_ref,
              y1_ref, gif_ref, gib_ref):
    CH = 2048
    for c in range(S * B // CH):
        sl = slice(c * CH, (c + 1) * CH)
        xc = x_ref[sl, :]
        gif_ref[sl, :] = _dot(xc, w1fi[:]) + b1fi[:]
        gib_ref[sl, :] = _dot(xc, w1bi[:]) + b1bi[:]

    def layer(whf, bhf, whb, bhb, dst_ref):
        def step(t, carry):
            hf, hb = carry
            tb = S - 1 - t
            ghf = _dot(hf, whf[:]) + bhf[:]
            hfn = _gates(gif_ref[pl.ds(t * B, B), :], ghf, hf)
            dst_ref[pl.ds(t * B, B), 0:H] = hfn
            ghb = _dot(hb, whb[:]) + bhb[:]
            hbn = _gates(gib_ref[pl.ds(tb * B, B), :], ghb, hb)
            dst_ref[pl.ds(tb * B, B), H:2 * H] = hbn
            return hfn, hbn

        h0 = jnp.zeros((B, H), F32)
        return jax.lax.fori_loop(0, S, step, (h0, h0))

    layer(w1fh, b1fh, w1bh, b1bh, y1_ref)
    for c in range(S * B // CH):
        sl = slice(c * CH, (c + 1) * CH)
        yc = y1_ref[sl, :]
        gif_ref[sl, :] = _dot(yc, w2fi[:]) + b2fi[:]
        gib_ref[sl, :] = _dot(yc, w2bi[:]) + b2bi[:]
    hf, hb = layer(w2fh, b2fh, w2bh, b2bh, out_ref)
    hf_ref[:] = hf
    hb_ref[:] = hb


def _encoder(x_tm, enc):
    args = [x_tm]
    for l in range(2):
        for d in ("f", "b"):
            w = enc[l][d]
            args += [w["Wi"].T, w["Wh"].T, w["bi"][None, :], w["bh"][None, :]]
    return pl.pallas_call(
        _enc_body,
        out_shape=[jax.ShapeDtypeStruct((S * B, 2 * H), F32),
                   jax.ShapeDtypeStruct((B, H), F32),
                   jax.ShapeDtypeStruct((B, H), F32)],
        scratch_shapes=[pltpu.VMEM((S * B, 2 * H), F32),
                        pltpu.VMEM((S * B, 3 * H), F32),
                        pltpu.VMEM((S * B, 3 * H), F32)],
    )(*args)


def _head_body(sums_ref, cnts_ref, hf_ref, hb_ref, h_ref,
               clng, clnb, cw, cb, vlng, vlnb, vw, vb,
               ctx_ref, rep_ref):
    sums = sums_ref[0] + sums_ref[1]
    cnts = cnts_ref[0] + cnts_ref[1]
    z_g = sums / jnp.maximum(cnts, 1.0)
    z = jnp.concatenate([z_g, hf_ref[:], hb_ref[:]], axis=1)
    zn = _ln(z, clng[:], clnb[:])
    ctx_ref[:] = _leaky(_dot(zn, cw[:]) + cb[:], 0.01)
    CH = 2048
    for c in range(S * B // CH):
        sl = slice(c * CH, (c + 1) * CH)
        hn = _ln(h_ref[sl, :], vlng[:], vlnb[:])
        rep_ref[sl, :] = _leaky(_dot(hn, vw[:]) + vb[:], 0.01)


def _head(sums_p, cnts_p, hf, hb, h_tm, p):
    return pl.pallas_call(
        _head_body,
        out_shape=[jax.ShapeDtypeStruct((B, 3 * H), F32),
                   jax.ShapeDtypeStruct((S * B, 2 * H), F32)],
    )(sums_p, cnts_p, hf, hb, h_tm,
      p["ctx_ln_g"][None, :], p["ctx_ln_b"][None, :],
      p["ctx_W"], p["ctx_b"][None, :],
      p["vec_ln_g"][None, :], p["vec_ln_b"][None, :],
      p["vec_W"], p["vec_b"][None, :])


def _dec_body(ctx_ref, rep_ref, emb_ref,
              w0i, w0h, b0i, b0h, w1i, w1h, b1i, b1h, lw, lb,
              out_ref, gi0_ref, y_ref):
    gi0_ref[0:B, :] = _dot(ctx_ref[:], w0i[:]) + b0i[:]
    CH = 2016
    for c in range(4):
        sl = slice(c * CH, (c + 1) * CH)
        osl = slice(B + c * CH, B + (c + 1) * CH)
        gi0_ref[osl, :] = (_dot(rep_ref[sl, :], w0i[0:2 * H, :])
                           + _dot(emb_ref[sl, :], w0i[2 * H:3 * H, :])
                           + b0i[:])

    def step(t, carry):
        h0, h1 = carry
        gh0 = _dot(h0, w0h[:]) + b0h[:]
        h0n = _gates(gi0_ref[pl.ds(t * B, B), :], gh0, h0)
        gi1 = _dot(h0n, w1i[:]) + b1i[:]
        gh1 = _dot(h1, w1h[:]) + b1h[:]
        h1n = _gates(gi1, gh1, h1)
        y_ref[pl.ds(t * B, B), :] = h1n
        return h0n, h1n

    hz = jnp.zeros((B, H), F32)
    jax.lax.fori_loop(0, S, step, (hz, hz))
    for c in range(4):
        sl = slice(c * 2048, (c + 1) * 2048)
        out_ref[sl, :] = _dot(y_ref[sl, :], lw[:]) + lb[:]


def _decoder(ctx, rep, emb, d):
    l0, l1 = d["layers"]
    dout = d["lin_W"].shape[1]
    return pl.pallas_call(
        _dec_body,
        out_shape=jax.ShapeDtypeStruct((S * B, dout), F32),
        scratch_shapes=[pltpu.VMEM((S * B, 3 * H), F32),
                        pltpu.VMEM((S * B, H), F32)],
    )(ctx, rep, emb,
      l0["Wi"].T, l0["Wh"].T, l0["bi"][None, :], l0["bh"][None, :],
      l1["Wi"].T, l1["Wh"].T, l1["bi"][None, :], l1["bh"][None, :],
      d["lin_W"], d["lin_b"][None, :])


def _sc_gather_body(og_ref, ap_ref, bg_ref, ones_ref, zeros_ref,
                    mapped_ref, sums_ref, cnts_ref,
                    ap_v, idx_v, rows_v, gsl_v, bidx_v, ones_v,
                    sh_sums, sh_cnts, sem):
    c = lax.axis_index("c")
    s = lax.axis_index("s")
    nc = 2
    wid = s * nc + c

    @pl.when(s == 0)
    def _zero():
        pltpu.sync_copy(zeros_ref, sh_sums)
        pltpu.sync_copy(zeros_ref, sh_cnts)

    plsc.subcore_barrier()

    pltpu.sync_copy(ap_ref.at[pl.ds(wid * 256, 256)], ap_v)
    for j in range(2):
        for i in range(8):
            v = ap_v[pl.ds(j * 128 + i * 16, 16)]
            idx_v[j, pl.ds(i * 16, 16)] = jnp.where(
                v >= 0, v, jnp.full((16,), N_NODES, jnp.int32))
    for j in range(2):
        pltpu.async_copy(og_ref.at[idx_v.at[j]],
                         rows_v.at[pl.ds(j * 128, 128)], sem).wait()
    pltpu.sync_copy(rows_v, mapped_ref.at[pl.ds(wid * 256, 256)])

    pltpu.sync_copy(og_ref.at[pl.ds(wid * 128, 128)], gsl_v)
    pltpu.sync_copy(bg_ref.at[pl.ds(wid * 128, 128)], bidx_v)
    pltpu.sync_copy(ones_ref, ones_v)
    pltpu.sync_copy(gsl_v, sh_sums.at[bidx_v], add=True)
    pltpu.sync_copy(ones_v, sh_cnts.at[bidx_v], add=True)
    plsc.subcore_barrier()

    @pl.when(s == 0)
    def _flush():
        pltpu.sync_copy(sh_sums, sums_ref.at[c])
        pltpu.sync_copy(sh_cnts, cnts_ref.at[c])


def _sc_gather_segsum(og_ext, ap_flat, batch_g):
    ones = jnp.ones((B, H), F32)
    zeros = jnp.zeros((B, H), F32)
    mesh = plsc.VectorSubcoreMesh(core_axis_name="c", subcore_axis_name="s")
    fn = pl.kernel(
        _sc_gather_body,
        mesh=mesh,
        out_type=[jax.ShapeDtypeStruct((S * B, H), F32),
                  jax.ShapeDtypeStruct((2, B, H), F32),
                  jax.ShapeDtypeStruct((2, B, H), F32)],
        scratch_types=[pltpu.VMEM((256,), jnp.int32),
                       pltpu.VMEM((2, 128), jnp.int32),
                       pltpu.VMEM((256, H), F32),
                       pltpu.VMEM((128, H), F32),
                       pltpu.VMEM((128,), jnp.int32),
                       pltpu.VMEM((B, H), F32),
                       pltpu.VMEM_SHARED((B, H), F32),
                       pltpu.VMEM_SHARED((B, H), F32),
                       pltpu.SemaphoreType.DMA],
    )
    return fn(og_ext, ap_flat, batch_g, ones, zeros)


E2 = 160


def _sc_gat_body(xp0_ref, xp1_ref, as0_ref, as1_ref, ad0_ref, ad1_ref,
                 src_ref, dst_ref, z256_ref,
                 outp0_ref, outp1_ref, denp_ref,
                 as0_v, as1_v, ad0_v, ad1_v, src_v, dst_v, sidx_v, didx_v,
                 ee0_v, ee1_v, eer_v, rows0_v, rows1_v,
                 sh_out0, sh_out1, sem):
    c = lax.axis_index("c")
    s = lax.axis_index("s")
    wid = s * 2 + c
    pltpu.sync_copy(z256_ref, sh_out0.at[pl.ds(s * 256, 256)])
    pltpu.sync_copy(z256_ref, sh_out1.at[pl.ds(s * 256, 256)])
    pltpu.sync_copy(as0_ref, as0_v)
    pltpu.sync_copy(as1_ref, as1_v)
    pltpu.sync_copy(ad0_ref, ad0_v)
    pltpu.sync_copy(ad1_ref, ad1_v)
    pltpu.sync_copy(src_ref.at[wid], src_v)
    pltpu.sync_copy(dst_ref.at[wid], dst_v)

    def split7(v):
        return [lax.shift_right_logical(v, 7), jnp.bitwise_and(v, 127)]

    zv = jnp.zeros((16,), F32)
    for r in range(64):
        for vv in range(8):
            eer_v[r, pl.ds(vv * 16, 16)] = zv
    plsc.subcore_barrier()

    for j in range(10):
        for i in range(4):
            sv = src_v[j, pl.ds(i * 16, 16)]
            dv = dst_v[j, pl.ds(i * 16, 16)]
            didx_v[pl.ds(i * 16, 16)] = dv
            e0 = (plsc.load_gather(as0_v, split7(sv))
                  + plsc.load_gather(ad0_v, split7(dv)))
            ee0 = jnp.exp(jnp.where(e0 >= 0.0, e0, 0.2 * e0))
            e1 = (plsc.load_gather(as1_v, split7(sv))
                  + plsc.load_gather(ad1_v, split7(dv)))
            ee1 = jnp.exp(jnp.where(e1 >= 0.0, e1, 0.2 * e1))
            ee0_v[j * 4 + i] = ee0
            ee1_v[j * 4 + i] = ee1
            ridx = lax.iota(jnp.int32, 16) + i * 16
            plsc.store_scatter(eer_v, [ridx, jnp.zeros((16,), jnp.int32)], ee0)
            plsc.store_scatter(eer_v, [ridx, jnp.ones((16,), jnp.int32)], ee1)
        pltpu.sync_copy(eer_v, sh_out0.at[didx_v], add=True)
    plsc.subcore_barrier()
    pltpu.sync_copy(sh_out0.at[pl.ds(s * 256, 256)],
                    denp_ref.at[c, pl.ds(s * 256, 256)])
    pltpu.sync_copy(z256_ref, sh_out0.at[pl.ds(s * 256, 256)])
    plsc.subcore_barrier()

    for j in range(10):
        for i in range(4):
            sidx_v[pl.ds(i * 16, 16)] = src_v[j, pl.ds(i * 16, 16)]
            didx_v[pl.ds(i * 16, 16)] = dst_v[j, pl.ds(i * 16, 16)]
        pltpu.async_copy(xp0_ref.at[sidx_v], rows0_v, sem).wait()
        pltpu.async_copy(xp1_ref.at[sidx_v], rows1_v, sem).wait()

        def scale(e, carry):
            gi = jnp.full((16,), j * 64 + e, jnp.int32)
            gidx = [lax.shift_right_logical(gi, 4), jnp.bitwise_and(gi, 15)]
            f0 = plsc.load_gather(ee0_v, gidx)
            f1 = plsc.load_gather(ee1_v, gidx)
            for v in range(8):
                rows0_v[e, pl.ds(v * 16, 16)] = (
                    rows0_v[e, pl.ds(v * 16, 16)] * f0)
                rows1_v[e, pl.ds(v * 16, 16)] = (
                    rows1_v[e, pl.ds(v * 16, 16)] * f1)
            return carry

        lax.fori_loop(0, 64, scale, 0)
        pltpu.sync_copy(rows0_v, sh_out0.at[didx_v], add=True)
        pltpu.sync_copy(rows1_v, sh_out1.at[didx_v], add=True)

    plsc.subcore_barrier()
    pltpu.sync_copy(sh_out0.at[pl.ds(s * 256, 256)],
                    outp0_ref.at[c, pl.ds(s * 256, 256)])
    pltpu.sync_copy(sh_out1.at[pl.ds(s * 256, 256)],
                    outp1_ref.at[c, pl.ds(s * 256, 256)])


def _sc_gat(xp0, xp1, as0, as1, ad0, ad1, src2, dst2):
    z256 = jnp.zeros((256, H), F32)
    mesh = plsc.VectorSubcoreMesh(core_axis_name="c", subcore_axis_name="s")
    fn = pl.kernel(
        _sc_gat_body,
        mesh=mesh,
        compiler_params=pltpu.CompilerParams(needs_layout_passes=False),
        out_type=[jax.ShapeDtypeStruct((2, N_NODES, H), F32),
                  jax.ShapeDtypeStruct((2, N_NODES, H), F32),
                  jax.ShapeDtypeStruct((2, N_NODES, H), F32)],
        scratch_types=[pltpu.VMEM((32, 128), F32),
                       pltpu.VMEM((32, 128), F32),
                       pltpu.VMEM((32, 128), F32),
                       pltpu.VMEM((32, 128), F32),
                       pltpu.VMEM((10, 64), jnp.int32),
                       pltpu.VMEM((10, 64), jnp.int32),
                       pltpu.VMEM((64,), jnp.int32),
                       pltpu.VMEM((64,), jnp.int32),
                       pltpu.VMEM((40, 16), F32),
                       pltpu.VMEM((40, 16), F32),
                       pltpu.VMEM((64, H), F32),
                       pltpu.VMEM((64, H), F32),
                       pltpu.VMEM((64, H), F32),
                       pltpu.VMEM_SHARED((N_NODES, H), F32),
                       pltpu.VMEM_SHARED((N_NODES, H), F32),
                       pltpu.SemaphoreType.DMA],
    )
    r2 = lambda a: a.reshape(32, 128)
    return fn(xp0, xp1, r2(as0), r2(as1), r2(ad0), r2(ad1), src2, dst2, z256)


def _gat_prep_tail(xp, acat, dcat, xp0_ref, xp1_ref,
                   as0_ref, as1_ref, ad0_ref, ad1_ref):
    ps = xp * acat
    pd = xp * dcat
    xp0_ref[:] = xp[:, :H]
    xp1_ref[:] = xp[:, H:]
    as0_ref[:] = jnp.sum(ps[:, :H], axis=1)
    as1_ref[:] = jnp.sum(ps[:, H:], axis=1)
    ad0_ref[:] = jnp.sum(pd[:, :H], axis=1)
    ad1_ref[:] = jnp.sum(pd[:, H:], axis=1)


_GAT_PREP_OUT = [jax.ShapeDtypeStruct((N_NODES, H), F32),
                 jax.ShapeDtypeStruct((N_NODES, H), F32),
                 jax.ShapeDtypeStruct((N_NODES,), F32),
                 jax.ShapeDtypeStruct((N_NODES,), F32),
                 jax.ShapeDtypeStruct((N_NODES,), F32),
                 jax.ShapeDtypeStruct((N_NODES,), F32)]


def _gat_prep_body(x_ref, w_ref, acat_ref, dcat_ref,
                   xp0_ref, xp1_ref, as0_ref, as1_ref, ad0_ref, ad1_ref):
    xp = _dot(x_ref[:], w_ref[:])
    _gat_prep_tail(xp, acat_ref[:], dcat_ref[:],
                   xp0_ref, xp1_ref, as0_ref, as1_ref, ad0_ref, ad1_ref)


def _gat_prep(x, W, acat, dcat):
    return pl.pallas_call(
        _gat_prep_body, out_shape=_GAT_PREP_OUT)(x, W, acat, dcat)


def _gat_normalize(outp0, outp1, denp, b):
    den = denp[0] + denp[1]
    d0 = den[:, 0:1] + 1e-16
    d1 = den[:, 1:2] + 1e-16
    o = jnp.concatenate([(outp0[0] + outp0[1]) / d0,
                         (outp1[0] + outp1[1]) / d1], axis=1)
    return _leaky(o + b, 0.01)


def _gat_comb_prep_body(outp0_ref, outp1_ref, denp_ref, b_ref, w_ref,
                        acat_ref, dcat_ref,
                        xp0_ref, xp1_ref, as0_ref, as1_ref, ad0_ref, ad1_ref):
    hx = _gat_normalize(outp0_ref[...], outp1_ref[...], denp_ref[...], b_ref[:])
    xp = _dot(hx, w_ref[:])
    _gat_prep_tail(xp, acat_ref[:], dcat_ref[:],
                   xp0_ref, xp1_ref, as0_ref, as1_ref, ad0_ref, ad1_ref)


def _gat_comb_prep(outp0, outp1, denp, b, W, acat, dcat):
    return pl.pallas_call(
        _gat_comb_prep_body, out_shape=_GAT_PREP_OUT)(
            outp0, outp1, denp, b, W, acat, dcat)


def _gat_final_body(outp0_ref, outp1_ref, denp_ref, b_ref, pw_ref, pb_ref,
                    og_ref):
    h2 = _gat_normalize(outp0_ref[...], outp1_ref[...], denp_ref[...], b_ref[:])
    og_ref[:] = _dot(h2, pw_ref[:]) + pb_ref[:]


def _gat_final(outp0, outp1, denp, b, pw, pb):
    return pl.pallas_call(
        _gat_final_body,
        out_shape=jax.ShapeDtypeStruct((N_NODES, H), F32))(
            outp0, outp1, denp, b, pw, pb)


def _posenc(seq_len, d):
    positions = jnp.arange(seq_len, dtype=F32)[:, None]
    div = jnp.exp(-jnp.log(10000.0) * jnp.arange(0, d, 2, dtype=F32) / d)
    ang = positions * div[None, :]
    pe = jnp.zeros((seq_len, d), dtype=F32)
    pe = pe.at[:, 0::2].set(jnp.sin(ang))
    pe = pe.at[:, 1::2].set(jnp.cos(ang))
    return pe


def kernel(Xg, Xs, Xa, edge_index, Act_pos, batch_g, params):
    p = params
    N = Xg.shape[0]
    loop = jnp.arange(N)
    src2 = jnp.concatenate([edge_index[0], loop]).astype(jnp.int32).reshape(32, 10, 64)
    dst2 = jnp.concatenate([edge_index[1], loop]).astype(jnp.int32).reshape(32, 10, 64)
    x_emb = p["acts_emb"][Xg[:, 0]]

    def _acat(a):
        return jnp.concatenate([a[0], a[1]])[None, :]

    xpa0, xpa1, as0, as1, ad0, ad1 = _gat_prep(
        x_emb, p["gat1_W"], _acat(p["gat1_asrc"]), _acat(p["gat1_adst"]))
    op10, op11, denp1 = _sc_gat(xpa0, xpa1, as0, as1, ad0, ad1, src2, dst2)
    xpb0, xpb1, bs0, bs1, bd0, bd1 = _gat_comb_prep(
        op10, op11, denp1, p["gat1_b"][None, :], p["gat2_W"],
        _acat(p["gat2_asrc"]), _acat(p["gat2_adst"]))
    op20, op21, denp2 = _sc_gat(xpb0, xpb1, bs0, bs1, bd0, bd1, src2, dst2)
    out_g = _gat_final(op20, op21, denp2, p["gat2_b"][None, :],
                       p["proj_W"], p["proj_b"][None, :])
    og_ext = jnp.concatenate([out_g, jnp.zeros((8, H), F32)], axis=0)

    ap = Act_pos.T
    mapped_f, sums_p, cnts_p = _sc_gather_segsum(
        og_ext, ap.reshape(S * B).astype(jnp.int32), batch_g.astype(jnp.int32))
    mapped = mapped_f.reshape(S, B, H)

    ids0 = Xs[:, :, 0].T
    ids1 = Xs[:, :, 1].T
    x_tm = jnp.concatenate([p["attr_emb_0"][ids0], p["attr_emb_1"][ids1]],
                           axis=-1).reshape(S * B, 2 * H)
    out_s_tm, hf, hb = _encoder(x_tm, p["enc_gru"])

    valid = ap >= 0
    pos_tm = _posenc(S, H)[:, None, :] * valid[:, :, None].astype(F32)
    h_tm = jnp.concatenate([out_s_tm.reshape(S, B, 2 * H), mapped, pos_tm],
                           axis=-1).reshape(S * B, 4 * H)
    ctx, rep = _head(sums_p, cnts_p, hf, hb, h_tm, p)

    outs = []
    for i in range(3):
        ids = (Xa if i == 0 else Xs[:, :, i - 1]).T[:S - 1]
        emb = p["emb_tabs"][i][ids].reshape((S - 1) * B, H)
        y = _decoder(ctx, rep, emb, p["decs"][i])
        outs.append(jnp.swapaxes(y.reshape(S, B, -1), 0, 1))
    return tuple(outs)


# speedup vs baseline: 7.8850x; 1.0006x over previous
import jax
import jax.numpy as jnp
from jax import lax
from jax.experimental import pallas as pl
from jax.experimental.pallas import tpu as pltpu
from jax.experimental.pallas import tpu_sc as plsc

H = 128
B = 128
S = 64
N_NODES = 4096
F32 = jnp.float32


def _leaky(x, s):
    return jnp.where(x >= 0, x, s * x)


def _dot(a, b):
    return jnp.dot(a, b, preferred_element_type=jnp.float32)


def _gates(gi, gh, h):
    r = jax.nn.sigmoid(gi[:, :H] + gh[:, :H])
    z = jax.nn.sigmoid(gi[:, H:2 * H] + gh[:, H:2 * H])
    n = jnp.tanh(gi[:, 2 * H:] + r * gh[:, 2 * H:])
    return (1.0 - z) * n + z * h


def _ln(x, g, b):
    m = jnp.mean(x, axis=1, keepdims=True)
    v = jnp.mean((x - m) ** 2, axis=1, keepdims=True)
    return (x - m) / jnp.sqrt(v + 1e-5) * g + b


def _enc_body(x_ref,
              w1fi, w1fh, b1fi, b1fh,
              w1bi, w1bh, b1bi, b1bh,
              w2fi, w2fh, b2fi, b2fh,
              w2bi, w2bh, b2bi, b2bh,
              out_ref, hf_ref, hb_ref,
              y1_ref, gif_ref, gib_ref):
    CH = 2048
    for c in range(S * B // CH):
        sl = slice(c * CH, (c + 1) * CH)
        xc = x_ref[sl, :]
        gif_ref[sl, :] = _dot(xc, w1fi[:]) + b1fi[:]
        gib_ref[sl, :] = _dot(xc, w1bi[:]) + b1bi[:]

    def layer(whf, bhf, whb, bhb, dst_ref):
        def step(t, carry):
            hf, hb = carry
            tb = S - 1 - t
            ghf = _dot(hf, whf[:]) + bhf[:]
            hfn = _gates(gif_ref[pl.ds(t * B, B), :], ghf, hf)
            dst_ref[pl.ds(t * B, B), 0:H] = hfn
            ghb = _dot(hb, whb[:]) + bhb[:]
            hbn = _gates(gib_ref[pl.ds(tb * B, B), :], ghb, hb)
            dst_ref[pl.ds(tb * B, B), H:2 * H] = hbn
            return hfn, hbn

        h0 = jnp.zeros((B, H), F32)
        return jax.lax.fori_loop(0, S, step, (h0, h0))

    layer(w1fh, b1fh, w1bh, b1bh, y1_ref)
    for c in range(S * B // CH):
        sl = slice(c * CH, (c + 1) * CH)
        yc = y1_ref[sl, :]
        gif_ref[sl, :] = _dot(yc, w2fi[:]) + b2fi[:]
        gib_ref[sl, :] = _dot(yc, w2bi[:]) + b2bi[:]
    hf, hb = layer(w2fh, b2fh, w2bh, b2bh, out_ref)
    hf_ref[:] = hf
    hb_ref[:] = hb


def _encoder(x_tm, enc):
    args = [x_tm]
    for l in range(2):
        for d in ("f", "b"):
            w = enc[l][d]
            args += [w["Wi"].T, w["Wh"].T, w["bi"][None, :], w["bh"][None, :]]
    return pl.pallas_call(
        _enc_body,
        out_shape=[jax.ShapeDtypeStruct((S * B, 2 * H), F32),
                   jax.ShapeDtypeStruct((B, H), F32),
                   jax.ShapeDtypeStruct((B, H), F32)],
        scratch_shapes=[pltpu.VMEM((S * B, 2 * H), F32),
                        pltpu.VMEM((S * B, 3 * H), F32),
                        pltpu.VMEM((S * B, 3 * H), F32)],
    )(*args)


def _head_body(sums_ref, cnts_ref, hf_ref, hb_ref, h_ref,
               clng, clnb, cw, cb, vlng, vlnb, vw, vb,
               ctx_ref, rep_ref):
    sums = sums_ref[0] + sums_ref[1]
    cnts = cnts_ref[0] + cnts_ref[1]
    z_g = sums / jnp.maximum(cnts, 1.0)
    z = jnp.concatenate([z_g, hf_ref[:], hb_ref[:]], axis=1)
    zn = _ln(z, clng[:], clnb[:])
    ctx_ref[:] = _leaky(_dot(zn, cw[:]) + cb[:], 0.01)
    CH = 2048
    for c in range(S * B // CH):
        sl = slice(c * CH, (c + 1) * CH)
        hn = _ln(h_ref[sl, :], vlng[:], vlnb[:])
        rep_ref[sl, :] = _leaky(_dot(hn, vw[:]) + vb[:], 0.01)


def _head(sums_p, cnts_p, hf, hb, h_tm, p):
    return pl.pallas_call(
        _head_body,
        out_shape=[jax.ShapeDtypeStruct((B, 3 * H), F32),
                   jax.ShapeDtypeStruct((S * B, 2 * H), F32)],
    )(sums_p, cnts_p, hf, hb, h_tm,
      p["ctx_ln_g"][None, :], p["ctx_ln_b"][None, :],
      p["ctx_W"], p["ctx_b"][None, :],
      p["vec_ln_g"][None, :], p["vec_ln_b"][None, :],
      p["vec_W"], p["vec_b"][None, :])


def _dec_body(ctx_ref, rep_ref, emb_ref,
              w0i, w0h, b0i, b0h, w1i, w1h, b1i, b1h, lw, lb,
              out_ref, gi0_ref, y_ref):
    gi0_ref[0:B, :] = _dot(ctx_ref[:], w0i[:]) + b0i[:]
    CH = 2016
    for c in range(4):
        sl = slice(c * CH, (c + 1) * CH)
        osl = slice(B + c * CH, B + (c + 1) * CH)
        gi0_ref[osl, :] = (_dot(rep_ref[sl, :], w0i[0:2 * H, :])
                           + _dot(emb_ref[sl, :], w0i[2 * H:3 * H, :])
                           + b0i[:])

    def step(t, carry):
        h0, h1 = carry
        gh0 = _dot(h0, w0h[:]) + b0h[:]
        h0n = _gates(gi0_ref[pl.ds(t * B, B), :], gh0, h0)
        gi1 = _dot(h0n, w1i[:]) + b1i[:]
        gh1 = _dot(h1, w1h[:]) + b1h[:]
        h1n = _gates(gi1, gh1, h1)
        y_ref[pl.ds(t * B, B), :] = h1n
        return h0n, h1n

    hz = jnp.zeros((B, H), F32)
    jax.lax.fori_loop(0, S, step, (hz, hz))
    for c in range(4):
        sl = slice(c * 2048, (c + 1) * 2048)
        out_ref[sl, :] = _dot(y_ref[sl, :], lw[:]) + lb[:]


def _decoder(ctx, rep, emb, d):
    l0, l1 = d["layers"]
    dout = d["lin_W"].shape[1]
    return pl.pallas_call(
        _dec_body,
        out_shape=jax.ShapeDtypeStruct((S * B, dout), F32),
        scratch_shapes=[pltpu.VMEM((S * B, 3 * H), F32),
                        pltpu.VMEM((S * B, H), F32)],
    )(ctx, rep, emb,
      l0["Wi"].T, l0["Wh"].T, l0["bi"][None, :], l0["bh"][None, :],
      l1["Wi"].T, l1["Wh"].T, l1["bi"][None, :], l1["bh"][None, :],
      d["lin_W"], d["lin_b"][None, :])


def _sc_gather_body(og_ref, ap_ref, bg_ref, ones_ref, zeros_ref,
                    mapped_ref, sums_ref, cnts_ref,
                    ap_v, idx_v, rows_v, gsl_v, bidx_v, ones_v,
                    sh_sums, sh_cnts, sem):
    c = lax.axis_index("c")
    s = lax.axis_index("s")
    nc = 2
    wid = s * nc + c

    @pl.when(s == 0)
    def _zero():
        pltpu.sync_copy(zeros_ref, sh_sums)
        pltpu.sync_copy(zeros_ref, sh_cnts)

    plsc.subcore_barrier()

    pltpu.sync_copy(ap_ref.at[pl.ds(wid * 256, 256)], ap_v)
    for j in range(2):
        for i in range(8):
            v = ap_v[pl.ds(j * 128 + i * 16, 16)]
            idx_v[j, pl.ds(i * 16, 16)] = jnp.where(
                v >= 0, v, jnp.full((16,), N_NODES, jnp.int32))
    for j in range(2):
        pltpu.async_copy(og_ref.at[idx_v.at[j]],
                         rows_v.at[pl.ds(j * 128, 128)], sem).wait()
    pltpu.sync_copy(rows_v, mapped_ref.at[pl.ds(wid * 256, 256)])

    pltpu.sync_copy(og_ref.at[pl.ds(wid * 128, 128)], gsl_v)
    pltpu.sync_copy(bg_ref.at[pl.ds(wid * 128, 128)], bidx_v)
    pltpu.sync_copy(ones_ref, ones_v)
    pltpu.sync_copy(gsl_v, sh_sums.at[bidx_v], add=True)
    pltpu.sync_copy(ones_v, sh_cnts.at[bidx_v], add=True)
    plsc.subcore_barrier()

    @pl.when(s == 0)
    def _flush():
        pltpu.sync_copy(sh_sums, sums_ref.at[c])
        pltpu.sync_copy(sh_cnts, cnts_ref.at[c])


def _sc_gather_segsum(og_ext, ap_flat, batch_g):
    ones = jnp.ones((B, H), F32)
    zeros = jnp.zeros((B, H), F32)
    mesh = plsc.VectorSubcoreMesh(core_axis_name="c", subcore_axis_name="s")
    fn = pl.kernel(
        _sc_gather_body,
        mesh=mesh,
        out_type=[jax.ShapeDtypeStruct((S * B, H), F32),
                  jax.ShapeDtypeStruct((2, B, H), F32),
                  jax.ShapeDtypeStruct((2, B, H), F32)],
        scratch_types=[pltpu.VMEM((256,), jnp.int32),
                       pltpu.VMEM((2, 128), jnp.int32),
                       pltpu.VMEM((256, H), F32),
                       pltpu.VMEM((128, H), F32),
                       pltpu.VMEM((128,), jnp.int32),
                       pltpu.VMEM((B, H), F32),
                       pltpu.VMEM_SHARED((B, H), F32),
                       pltpu.VMEM_SHARED((B, H), F32),
                       pltpu.SemaphoreType.DMA],
    )
    return fn(og_ext, ap_flat, batch_g, ones, zeros)


def _sc_gat_body(xp0_ref, xp1_ref, as0_ref, as1_ref, ad0_ref, ad1_ref,
                 src_ref, dst_ref, z256_ref,
                 outp0_ref, outp1_ref, denp_ref,
                 as0_v, as1_v, ad0_v, ad1_v, src_v, dst_v, sidx_v, didx_v,
                 ee0_v, ee1_v, eer_v, rows0_v, rows1_v,
                 sh_out0, sh_out1, sem):
    c = lax.axis_index("c")
    s = lax.axis_index("s")
    wid = s * 2 + c
    pltpu.sync_copy(z256_ref, sh_out0.at[pl.ds(s * 256, 256)])
    pltpu.sync_copy(z256_ref, sh_out1.at[pl.ds(s * 256, 256)])
    pltpu.sync_copy(as0_ref, as0_v)
    pltpu.sync_copy(as1_ref, as1_v)
    pltpu.sync_copy(ad0_ref, ad0_v)
    pltpu.sync_copy(ad1_ref, ad1_v)
    pltpu.sync_copy(src_ref.at[wid], src_v)
    pltpu.sync_copy(dst_ref.at[wid], dst_v)

    def split7(v):
        return [lax.shift_right_logical(v, 7), jnp.bitwise_and(v, 127)]

    zv = jnp.zeros((16,), F32)
    for r in range(64):
        for vv in range(8):
            eer_v[r, pl.ds(vv * 16, 16)] = zv
    plsc.subcore_barrier()

    for j in range(10):
        for i in range(4):
            sv = src_v[j, pl.ds(i * 16, 16)]
            dv = dst_v[j, pl.ds(i * 16, 16)]
            didx_v[pl.ds(i * 16, 16)] = dv
            e0 = (plsc.load_gather(as0_v, split7(sv))
                  + plsc.load_gather(ad0_v, split7(dv)))
            ee0 = jnp.exp(jnp.where(e0 >= 0.0, e0, 0.2 * e0))
            e1 = (plsc.load_gather(as1_v, split7(sv))
                  + plsc.load_gather(ad1_v, split7(dv)))
            ee1 = jnp.exp(jnp.where(e1 >= 0.0, e1, 0.2 * e1))
            ee0_v[j * 4 + i] = ee0
            ee1_v[j * 4 + i] = ee1
            ridx = lax.iota(jnp.int32, 16) + i * 16
            plsc.store_scatter(eer_v, [ridx, jnp.zeros((16,), jnp.int32)], ee0)
            plsc.store_scatter(eer_v, [ridx, jnp.ones((16,), jnp.int32)], ee1)
        pltpu.sync_copy(eer_v, sh_out0.at[didx_v], add=True)
    plsc.subcore_barrier()
    pltpu.sync_copy(sh_out0.at[pl.ds(s * 256, 256)],
                    denp_ref.at[c, pl.ds(s * 256, 256)])
    pltpu.sync_copy(z256_ref, sh_out0.at[pl.ds(s * 256, 256)])
    plsc.subcore_barrier()

    for j in range(10):
        for i in range(4):
            sidx_v[pl.ds(i * 16, 16)] = src_v[j, pl.ds(i * 16, 16)]
            didx_v[pl.ds(i * 16, 16)] = dst_v[j, pl.ds(i * 16, 16)]
        pltpu.async_copy(xp0_ref.at[sidx_v], rows0_v, sem).wait()
        pltpu.async_copy(xp1_ref.at[sidx_v], rows1_v, sem).wait()

        def scale(e, carry):
            gi = jnp.full((16,), j * 64 + e, jnp.int32)
            gidx = [lax.shift_right_logical(gi, 4), jnp.bitwise_and(gi, 15)]
            f0 = plsc.load_gather(ee0_v, gidx)
            f1 = plsc.load_gather(ee1_v, gidx)
            for v in range(8):
                rows0_v[e, pl.ds(v * 16, 16)] = (
                    rows0_v[e, pl.ds(v * 16, 16)] * f0)
                rows1_v[e, pl.ds(v * 16, 16)] = (
                    rows1_v[e, pl.ds(v * 16, 16)] * f1)
            return carry

        lax.fori_loop(0, 64, scale, 0)
        pltpu.sync_copy(rows0_v, sh_out0.at[didx_v], add=True)
        pltpu.sync_copy(rows1_v, sh_out1.at[didx_v], add=True)

    plsc.subcore_barrier()
    pltpu.sync_copy(sh_out0.at[pl.ds(s * 256, 256)],
                    outp0_ref.at[c, pl.ds(s * 256, 256)])
    pltpu.sync_copy(sh_out1.at[pl.ds(s * 256, 256)],
                    outp1_ref.at[c, pl.ds(s * 256, 256)])


def _sc_gat(xp0, xp1, as0, as1, ad0, ad1, src2, dst2):
    z256 = jnp.zeros((256, H), F32)
    mesh = plsc.VectorSubcoreMesh(core_axis_name="c", subcore_axis_name="s")
    fn = pl.kernel(
        _sc_gat_body,
        mesh=mesh,
        compiler_params=pltpu.CompilerParams(needs_layout_passes=False),
        out_type=[jax.ShapeDtypeStruct((2, N_NODES, H), F32),
                  jax.ShapeDtypeStruct((2, N_NODES, H), F32),
                  jax.ShapeDtypeStruct((2, N_NODES, H), F32)],
        scratch_types=[pltpu.VMEM((32, 128), F32),
                       pltpu.VMEM((32, 128), F32),
                       pltpu.VMEM((32, 128), F32),
                       pltpu.VMEM((32, 128), F32),
                       pltpu.VMEM((10, 64), jnp.int32),
                       pltpu.VMEM((10, 64), jnp.int32),
                       pltpu.VMEM((64,), jnp.int32),
                       pltpu.VMEM((64,), jnp.int32),
                       pltpu.VMEM((40, 16), F32),
                       pltpu.VMEM((40, 16), F32),
                       pltpu.VMEM((64, H), F32),
                       pltpu.VMEM((64, H), F32),
                       pltpu.VMEM((64, H), F32),
                       pltpu.VMEM_SHARED((N_NODES, H), F32),
                       pltpu.VMEM_SHARED((N_NODES, H), F32),
                       pltpu.SemaphoreType.DMA],
    )
    r2 = lambda a: a.reshape(32, 128)
    return fn(xp0, xp1, r2(as0), r2(as1), r2(ad0), r2(ad1), src2, dst2, z256)


def _gat_prep_tail(xp, acat, dcat, xp0_ref, xp1_ref,
                   as0_ref, as1_ref, ad0_ref, ad1_ref):
    ps = xp * acat
    pd = xp * dcat
    xp0_ref[:] = xp[:, :H]
    xp1_ref[:] = xp[:, H:]
    as0_ref[:] = jnp.sum(ps[:, :H], axis=1)
    as1_ref[:] = jnp.sum(ps[:, H:], axis=1)
    ad0_ref[:] = jnp.sum(pd[:, :H], axis=1)
    ad1_ref[:] = jnp.sum(pd[:, H:], axis=1)


_GAT_PREP_OUT = [jax.ShapeDtypeStruct((N_NODES, H), F32),
                 jax.ShapeDtypeStruct((N_NODES, H), F32),
                 jax.ShapeDtypeStruct((N_NODES,), F32),
                 jax.ShapeDtypeStruct((N_NODES,), F32),
                 jax.ShapeDtypeStruct((N_NODES,), F32),
                 jax.ShapeDtypeStruct((N_NODES,), F32)]


def _gat_prep_body(x_ref, w_ref, acat_ref, dcat_ref,
                   xp0_ref, xp1_ref, as0_ref, as1_ref, ad0_ref, ad1_ref):
    xp = _dot(x_ref[:], w_ref[:])
    _gat_prep_tail(xp, acat_ref[:], dcat_ref[:],
                   xp0_ref, xp1_ref, as0_ref, as1_ref, ad0_ref, ad1_ref)


def _gat_prep(x, W, acat, dcat):
    return pl.pallas_call(
        _gat_prep_body, out_shape=_GAT_PREP_OUT)(x, W, acat, dcat)


def _gat_normalize(outp0, outp1, denp, b):
    den = denp[0] + denp[1]
    d0 = den[:, 0:1] + 1e-16
    d1 = den[:, 1:2] + 1e-16
    o = jnp.concatenate([(outp0[0] + outp0[1]) / d0,
                         (outp1[0] + outp1[1]) / d1], axis=1)
    return _leaky(o + b, 0.01)


def _gat_comb_prep_body(outp0_ref, outp1_ref, denp_ref, b_ref, w_ref,
                        acat_ref, dcat_ref,
                        xp0_ref, xp1_ref, as0_ref, as1_ref, ad0_ref, ad1_ref):
    hx = _gat_normalize(outp0_ref[...], outp1_ref[...], denp_ref[...], b_ref[:])
    xp = _dot(hx, w_ref[:])
    _gat_prep_tail(xp, acat_ref[:], dcat_ref[:],
                   xp0_ref, xp1_ref, as0_ref, as1_ref, ad0_ref, ad1_ref)


def _gat_comb_prep(outp0, outp1, denp, b, W, acat, dcat):
    return pl.pallas_call(
        _gat_comb_prep_body, out_shape=_GAT_PREP_OUT)(
            outp0, outp1, denp, b, W, acat, dcat)


def _gat_final_body(outp0_ref, outp1_ref, denp_ref, b_ref, pw_ref, pb_ref,
                    og_ref):
    h2 = _gat_normalize(outp0_ref[...], outp1_ref[...], denp_ref[...], b_ref[:])
    og_ref[:] = _dot(h2, pw_ref[:]) + pb_ref[:]


def _gat_final(outp0, outp1, denp, b, pw, pb):
    return pl.pallas_call(
        _gat_final_body,
        out_shape=jax.ShapeDtypeStruct((N_NODES, H), F32))(
            outp0, outp1, denp, b, pw, pb)


def _posenc(seq_len, d):
    positions = jnp.arange(seq_len, dtype=F32)[:, None]
    div = jnp.exp(-jnp.log(10000.0) * jnp.arange(0, d, 2, dtype=F32) / d)
    ang = positions * div[None, :]
    pe = jnp.zeros((seq_len, d), dtype=F32)
    pe = pe.at[:, 0::2].set(jnp.sin(ang))
    pe = pe.at[:, 1::2].set(jnp.cos(ang))
    return pe


def kernel(Xg, Xs, Xa, edge_index, Act_pos, batch_g, params):
    p = params
    N = Xg.shape[0]
    loop = jnp.arange(N)
    src2 = jnp.concatenate([edge_index[0], loop]).astype(jnp.int32).reshape(32, 10, 64)
    dst2 = jnp.concatenate([edge_index[1], loop]).astype(jnp.int32).reshape(32, 10, 64)
    x_emb = p["acts_emb"][Xg[:, 0]]

    def _acat(a):
        return jnp.concatenate([a[0], a[1]])[None, :]

    xpa0, xpa1, as0, as1, ad0, ad1 = _gat_prep(
        x_emb, p["gat1_W"], _acat(p["gat1_asrc"]), _acat(p["gat1_adst"]))
    op10, op11, denp1 = _sc_gat(xpa0, xpa1, as0, as1, ad0, ad1, src2, dst2)
    xpb0, xpb1, bs0, bs1, bd0, bd1 = _gat_comb_prep(
        op10, op11, denp1, p["gat1_b"][None, :], p["gat2_W"],
        _acat(p["gat2_asrc"]), _acat(p["gat2_adst"]))
    op20, op21, denp2 = _sc_gat(xpb0, xpb1, bs0, bs1, bd0, bd1, src2, dst2)
    out_g = _gat_final(op20, op21, denp2, p["gat2_b"][None, :],
                       p["proj_W"], p["proj_b"][None, :])
    og_ext = jnp.concatenate([out_g, jnp.zeros((8, H), F32)], axis=0)

    ap = Act_pos.T
    mapped_f, sums_p, cnts_p = _sc_gather_segsum(
        og_ext, ap.reshape(S * B).astype(jnp.int32), batch_g.astype(jnp.int32))
    mapped = mapped_f.reshape(S, B, H)

    ids0 = Xs[:, :, 0].T
    ids1 = Xs[:, :, 1].T
    x_tm = jnp.concatenate([p["attr_emb_0"][ids0], p["attr_emb_1"][ids1]],
                           axis=-1).reshape(S * B, 2 * H)
    out_s_tm, hf, hb = _encoder(x_tm, p["enc_gru"])

    valid = ap >= 0
    pos_tm = _posenc(S, H)[:, None, :] * valid[:, :, None].astype(F32)
    h_tm = jnp.concatenate([out_s_tm.reshape(S, B, 2 * H), mapped, pos_tm],
                           axis=-1).reshape(S * B, 4 * H)
    ctx, rep = _head(sums_p, cnts_p, hf, hb, h_tm, p)

    outs = []
    for i in range(3):
        ids = (Xa if i == 0 else Xs[:, :, i - 1]).T[:S - 1]
        emb = p["emb_tabs"][i][ids].reshape((S - 1) * B, H)
        y = _decoder(ctx, rep, emb, p["decs"][i])
        outs.append(jnp.swapaxes(y.reshape(S, B, -1), 0, 1))
    return tuple(outs)
